# Initial kernel scaffold; baseline (speedup 1.0000x reference)
import jax
import jax.numpy as jnp
from jax.experimental import pallas as pl


def kernel(x, segment_x, word_table, pos_emb, seg_table):
    raise NotImplementedError("write your pallas kernel here")


# speedup vs baseline: 1.2539x; 1.2539x over previous
import jax
import jax.numpy as jnp
from jax import lax
from jax.experimental import pallas as pl
from jax.experimental.pallas import tpu as pltpu
from jax.experimental.pallas import tpu_sc as plsc

B, L, DIM = 1024, 200, 64
SEG = 3
NC, NS, LANES = 2, 16, 16
NW = NC * NS
N = B * L
PER_W = N // NW
KI = 5
CH = KI * 128
G = PER_W // CH
def _body(xi_hbm, si_hbm, word_hbm, comb_hbm, out_hbm,
          idx_v, cidx_v, rows_v, crows_v, sem_w, sem_c):
    c = lax.axis_index("c")
    s = lax.axis_index("s")
    wid = s * NC + c
    iota = lax.iota(jnp.int32, LANES)

    def chunk(g, carry):
        nbase = wid * PER_W + g * CH
        pltpu.sync_copy(xi_hbm.at[pl.ds(nbase, CH)], idx_v)
        pltpu.sync_copy(si_hbm.at[pl.ds(nbase, CH)], cidx_v)
        for j in range(CH // 16):
            sl = pl.ds(j * 16, 16)
            seg = cidx_v[sl]
            pos = lax.rem(iota + (nbase + j * 16), L)
            cidx_v[sl] = pos * SEG + seg
        descs = []
        for k in range(KI):
            isl = pl.ds(k * 128, 128)
            dst = pl.ds(k * 128, 128)
            descs.append(pltpu.async_copy(
                word_hbm.at[idx_v.at[isl]], rows_v.at[dst], sem_w))
            descs.append(pltpu.async_copy(
                comb_hbm.at[cidx_v.at[isl]], crows_v.at[dst], sem_c))
        for d in descs:
            d.wait()

        def add(r, carry2):
            for cc in range(DIM // 16):
                sl = pl.ds(cc * 16, 16)
                rows_v[r, sl] = rows_v[r, sl] + crows_v[r, sl]
            return carry2
        lax.fori_loop(0, CH, add, 0)
        pltpu.sync_copy(rows_v, out_hbm.at[pl.ds(wid * PER_W + g * CH, CH)])
        return carry

    lax.fori_loop(0, G, chunk, 0)


def kernel(x, segment_x, word_table, pos_emb, seg_table):
    xf = x.reshape(N).astype(jnp.int32)
    sf = segment_x.reshape(N).astype(jnp.int32)
    comb = (pos_emb[0, :L, :][:, None, :] + seg_table[None, :, :]
            ).reshape(L * SEG, DIM).astype(jnp.float32)
    mesh = plsc.VectorSubcoreMesh(core_axis_name="c", subcore_axis_name="s",
                                  num_cores=NC, num_subcores=NS)
    out = pl.kernel(
        _body,
        out_type=jax.ShapeDtypeStruct((N, DIM), jnp.float32),
        mesh=mesh,
        scratch_types=[
            pltpu.VMEM((CH,), jnp.int32),
            pltpu.VMEM((CH,), jnp.int32),
            pltpu.VMEM((CH, DIM), jnp.float32),
            pltpu.VMEM((CH, DIM), jnp.float32),
            pltpu.SemaphoreType.DMA,
            pltpu.SemaphoreType.DMA,
        ],
        compiler_params=pltpu.CompilerParams(use_tc_tiling_on_sc=False),
    )(xf, sf, word_table, comb)
    return out.reshape(B, L, DIM)
